# baseline (device time: 655390 ns/iter reference)
import jax
import jax.numpy as jnp
from jax import lax
from jax.experimental import pallas as pl
from jax.experimental.pallas import tpu as pltpu

N_DEV = 4
M_BLK = 1024
NB = 1024
HALF = NB // 2


def _gelu(y):
    c = 0.7978845608028654
    return 0.5 * y * (1.0 + jnp.tanh(c * (y + 0.044715 * y * y * y)))


def kernel(x, w_mat):
    m, k_local = x.shape
    _, n = w_mat.shape
    n_steps = n // NB

    def body(x_ref, w_ref, out_ref, comm_cw, comm_ccw,
             send_cw, recv_cw, send_ccw, recv_ccw):
        p = lax.axis_index("i")
        right = (p + 1) % N_DEV
        left = (p - 1) % N_DEV
        j = pl.program_id(0)

        @pl.when(j == 0)
        def _():
            bsem = pltpu.get_barrier_semaphore()
            for nbr in (left, right):
                pl.semaphore_signal(
                    bsem, inc=1,
                    device_id=(nbr,), device_id_type=pl.DeviceIdType.MESH,
                )
            pl.semaphore_wait(bsem, 2)

        def partial(chunk, lo):
            xb = x_ref[pl.ds(chunk * M_BLK, M_BLK), :]
            return jnp.dot(xb, w_ref[:, lo:lo + HALF],
                           preferred_element_type=jnp.float32)

        comm_cw[0, :, :] = partial((p - 1) % N_DEV, 0)
        comm_ccw[0, :, :] = partial((p + 1) % N_DEV, HALF)

        for s in range(N_DEV - 1):
            snd, rcv = s % 2, (s + 1) % 2
            rd_cw = pltpu.make_async_remote_copy(
                src_ref=comm_cw.at[snd], dst_ref=comm_cw.at[rcv],
                send_sem=send_cw.at[s], recv_sem=recv_cw.at[s],
                device_id=(right,), device_id_type=pl.DeviceIdType.MESH,
            )
            rd_ccw = pltpu.make_async_remote_copy(
                src_ref=comm_ccw.at[snd], dst_ref=comm_ccw.at[rcv],
                send_sem=send_ccw.at[s], recv_sem=recv_ccw.at[s],
                device_id=(left,), device_id_type=pl.DeviceIdType.MESH,
            )
            rd_cw.start()
            rd_ccw.start()

            c_cw = (p - 2 - s) % N_DEV
            c_ccw = (p + 2 + s) % N_DEV
            pcw = partial(c_cw, 0)
            pccw = partial(c_ccw, HALF)

            rd_cw.wait()
            rd_ccw.wait()
            if s < N_DEV - 2:
                comm_cw[rcv, :, :] = comm_cw[rcv, :, :] + pcw
                comm_ccw[rcv, :, :] = comm_ccw[rcv, :, :] + pccw
            else:
                out_ref[:, 0:HALF] = _gelu(comm_cw[rcv, :, :] + pcw)
                out_ref[:, HALF:NB] = _gelu(comm_ccw[rcv, :, :] + pccw)

    return pl.pallas_call(
        body,
        grid=(n_steps,),
        out_shape=jax.ShapeDtypeStruct((M_BLK, n), jnp.float32),
        in_specs=[
            pl.BlockSpec(memory_space=pltpu.VMEM),
            pl.BlockSpec((k_local, NB), lambda j: (0, j)),
        ],
        out_specs=pl.BlockSpec((M_BLK, NB), lambda j: (0, j)),
        scratch_shapes=[
            pltpu.VMEM((2, M_BLK, HALF), jnp.float32),
            pltpu.VMEM((2, M_BLK, HALF), jnp.float32),
            pltpu.SemaphoreType.DMA((N_DEV - 1,)),
            pltpu.SemaphoreType.DMA((N_DEV - 1,)),
            pltpu.SemaphoreType.DMA((N_DEV - 1,)),
            pltpu.SemaphoreType.DMA((N_DEV - 1,)),
        ],
        compiler_params=pltpu.CompilerParams(
            collective_id=0,
            dimension_semantics=("arbitrary",),
        ),
    )(x, w_mat)


# device time: 616254 ns/iter; 1.0635x vs baseline; 1.0635x over previous
import jax
import jax.numpy as jnp
from jax import lax
from jax.experimental import pallas as pl
from jax.experimental.pallas import tpu as pltpu

N_DEV = 4
M_BLK = 1024
NB = 1024
HALF = NB // 2
NSUB = 2
SUB = HALF // NSUB


def _gelu(y):
    c = 0.7978845608028654
    return 0.5 * y * (1.0 + jnp.tanh(c * (y + 0.044715 * y * y * y)))


def kernel(x, w_mat):
    m, k_local = x.shape
    _, n = w_mat.shape
    n_steps = n // NB

    def body(x_ref, w_ref, out_ref, comm_cw, comm_ccw,
             send_cw, recv_cw, send_ccw, recv_ccw):
        p = lax.axis_index("i")
        right = (p + 1) % N_DEV
        left = (p - 1) % N_DEV
        j = pl.program_id(0)

        @pl.when(j == 0)
        def _():
            bsem = pltpu.get_barrier_semaphore()
            for nbr in (left, right):
                pl.semaphore_signal(
                    bsem, inc=1,
                    device_id=(nbr,), device_id_type=pl.DeviceIdType.MESH,
                )
            pl.semaphore_wait(bsem, 2)

        def partial(chunk, lo):
            xb = x_ref[pl.ds(chunk * M_BLK, M_BLK), :]
            return jnp.dot(xb, w_ref[:, lo:lo + HALF],
                           preferred_element_type=jnp.float32)

        comm_cw[0, :, :] = partial((p - 1) % N_DEV, 0)
        comm_ccw[0, :, :] = partial((p + 1) % N_DEV, HALF)

        dirs = (
            (comm_cw, send_cw, recv_cw, right, 0),
            (comm_ccw, send_ccw, recv_ccw, left, HALF),
        )

        def mk(di, s, q):
            buf, ssem, rsem, dev, _ = dirs[di]
            snd, rcv = s % 2, (s + 1) % 2
            c0 = q * SUB
            return pltpu.make_async_remote_copy(
                src_ref=buf.at[snd, :, pl.ds(c0, SUB)],
                dst_ref=buf.at[rcv, :, pl.ds(c0, SUB)],
                send_sem=ssem.at[s * NSUB + q],
                recv_sem=rsem.at[s * NSUB + q],
                device_id=(dev,), device_id_type=pl.DeviceIdType.MESH,
            )

        started = []
        for di in (0, 1):
            for q in range(NSUB):
                r = mk(di, 0, q)
                r.start()
                started.append(r)

        for s in range(N_DEV - 1):
            ps = (
                partial((p - 2 - s) % N_DEV, 0),
                partial((p + 2 + s) % N_DEV, HALF),
            )
            rcv = (s + 1) % 2
            for q in range(NSUB):
                for di in (0, 1):
                    mk(di, s, q).wait_recv()
                    buf = dirs[di][0]
                    c0 = q * SUB
                    acc = buf[rcv, :, c0:c0 + SUB] + ps[di][:, c0:c0 + SUB]
                    if s < N_DEV - 2:
                        buf[rcv, :, c0:c0 + SUB] = acc
                        r = mk(di, s + 1, q)
                        r.start()
                        started.append(r)
                    else:
                        lo = dirs[di][4] + c0
                        out_ref[:, lo:lo + SUB] = _gelu(acc)

        for r in started:
            r.wait_send()

    return pl.pallas_call(
        body,
        grid=(n_steps,),
        out_shape=jax.ShapeDtypeStruct((M_BLK, n), jnp.float32),
        in_specs=[
            pl.BlockSpec(memory_space=pltpu.VMEM),
            pl.BlockSpec((k_local, NB), lambda j: (0, j)),
        ],
        out_specs=pl.BlockSpec((M_BLK, NB), lambda j: (0, j)),
        scratch_shapes=[
            pltpu.VMEM((2, M_BLK, HALF), jnp.float32),
            pltpu.VMEM((2, M_BLK, HALF), jnp.float32),
            pltpu.SemaphoreType.DMA(((N_DEV - 1) * NSUB,)),
            pltpu.SemaphoreType.DMA(((N_DEV - 1) * NSUB,)),
            pltpu.SemaphoreType.DMA(((N_DEV - 1) * NSUB,)),
            pltpu.SemaphoreType.DMA(((N_DEV - 1) * NSUB,)),
        ],
        compiler_params=pltpu.CompilerParams(
            collective_id=0,
            dimension_semantics=("arbitrary",),
        ),
    )(x, w_mat)


# device time: 332863 ns/iter; 1.9689x vs baseline; 1.8514x over previous
import jax
import jax.numpy as jnp
from jax import lax
from jax.experimental import pallas as pl
from jax.experimental.pallas import tpu as pltpu

N_DEV = 4
M_BLK = 1024
NB = 1024
HALF = NB // 2
NSUB = 2
SUB = HALF // NSUB


def _gelu(y):
    c = 0.7978845608028654
    return 0.5 * y * (1.0 + jnp.tanh(c * (y + 0.044715 * y * y * y)))


def kernel(x, w_mat):
    m, k_local = x.shape
    _, n = w_mat.shape
    n_steps = n // NB

    def body(x_ref, w_ref, out_ref, comm_cw, comm_ccw, fin_cw, fin_ccw,
             own_cw, own_ccw, send_cw, recv_cw, send_ccw, recv_ccw):
        p = lax.axis_index("i")
        right = (p + 1) % N_DEV
        left = (p - 1) % N_DEV
        j = pl.program_id(0)

        @pl.when(j == 0)
        def _():
            bsem = pltpu.get_barrier_semaphore()
            for nbr in (left, right):
                pl.semaphore_signal(
                    bsem, inc=1,
                    device_id=(nbr,), device_id_type=pl.DeviceIdType.MESH,
                )
            pl.semaphore_wait(bsem, 2)

        def partial(chunk, lo, c0=0, w=HALF):
            xb = x_ref[pl.ds(chunk * M_BLK, M_BLK), :]
            return jnp.dot(xb, w_ref[:, lo + c0:lo + c0 + w],
                           preferred_element_type=jnp.float32)

        dirs = (
            (comm_cw, send_cw, recv_cw, right, 0, fin_cw, own_cw),
            (comm_ccw, send_ccw, recv_ccw, left, HALF, fin_ccw, own_ccw),
        )

        def mk(di, s, q):
            buf, ssem, rsem, dev, _, fin, _ = dirs[di]
            snd, rcv = s % 2, (s + 1) % 2
            c0 = q * SUB
            if s == N_DEV - 2:
                dst = fin.at[:, pl.ds(c0, SUB)]
            else:
                dst = buf.at[rcv, :, pl.ds(c0, SUB)]
            return pltpu.make_async_remote_copy(
                src_ref=buf.at[snd, :, pl.ds(c0, SUB)],
                dst_ref=dst,
                send_sem=ssem.at[s * NSUB + q],
                recv_sem=rsem.at[s * NSUB + q],
                device_id=(dev,), device_id_type=pl.DeviceIdType.MESH,
            )

        def consume_prev():
            for di in (0, 1):
                lo, fin, own = dirs[di][4], dirs[di][5], dirs[di][6]
                for q in range(NSUB):
                    mk(di, N_DEV - 2, q).wait_recv()
                    c0 = q * SUB
                    out_ref[:, lo + c0:lo + c0 + SUB] = _gelu(
                        fin[:, c0:c0 + SUB].astype(jnp.float32)
                        + own[:, c0:c0 + SUB].astype(jnp.float32)
                    )

        @pl.when(j < n_steps)
        def _ring():
            started = []
            for q in range(NSUB):
                for di in (0, 1):
                    buf, lo = dirs[di][0], dirs[di][4]
                    seed_c = ((p - 1) % N_DEV, (p + 1) % N_DEV)[di]
                    c0 = q * SUB
                    buf[0, :, c0:c0 + SUB] = partial(
                        seed_c, lo, c0, SUB).astype(jnp.bfloat16)
                    r = mk(di, 0, q)
                    r.start()
                    started.append(r)

            @pl.when(j > 0)
            def _():
                consume_prev()

            for s in range(N_DEV - 1):
                ps = (
                    partial((p - 2 - s) % N_DEV, 0),
                    partial((p + 2 + s) % N_DEV, HALF),
                )
                rcv = (s + 1) % 2
                if s < N_DEV - 2:
                    for q in range(NSUB):
                        for di in (0, 1):
                            mk(di, s, q).wait_recv()
                            buf = dirs[di][0]
                            c0 = q * SUB
                            buf[rcv, :, c0:c0 + SUB] = (
                                buf[rcv, :, c0:c0 + SUB].astype(jnp.float32)
                                + ps[di][:, c0:c0 + SUB]
                            ).astype(jnp.bfloat16)
                            r = mk(di, s + 1, q)
                            r.start()
                            started.append(r)
                else:
                    for di in (0, 1):
                        dirs[di][6][:, :] = ps[di].astype(jnp.bfloat16)

            for r in started:
                r.wait_send()

        @pl.when(j == n_steps)
        def _tail():
            consume_prev()

    return pl.pallas_call(
        body,
        grid=(n_steps + 1,),
        out_shape=jax.ShapeDtypeStruct((M_BLK, n), jnp.float32),
        in_specs=[
            pl.BlockSpec(memory_space=pltpu.VMEM),
            pl.BlockSpec((k_local, NB),
                         lambda j: (0, jnp.minimum(j, n_steps - 1))),
        ],
        out_specs=pl.BlockSpec((M_BLK, NB),
                               lambda j: (0, jnp.maximum(j - 1, 0))),
        scratch_shapes=[
            pltpu.VMEM((2, M_BLK, HALF), jnp.bfloat16),
            pltpu.VMEM((2, M_BLK, HALF), jnp.bfloat16),
            pltpu.VMEM((M_BLK, HALF), jnp.bfloat16),
            pltpu.VMEM((M_BLK, HALF), jnp.bfloat16),
            pltpu.VMEM((M_BLK, HALF), jnp.bfloat16),
            pltpu.VMEM((M_BLK, HALF), jnp.bfloat16),
            pltpu.SemaphoreType.DMA(((N_DEV - 1) * NSUB,)),
            pltpu.SemaphoreType.DMA(((N_DEV - 1) * NSUB,)),
            pltpu.SemaphoreType.DMA(((N_DEV - 1) * NSUB,)),
            pltpu.SemaphoreType.DMA(((N_DEV - 1) * NSUB,)),
        ],
        compiler_params=pltpu.CompilerParams(
            collective_id=0,
            dimension_semantics=("arbitrary",),
        ),
    )(x, w_mat)
